# baseline (speedup 1.0000x reference)
import functools

import jax
import jax.numpy as jnp
from jax import lax
from jax.experimental import pallas as pl
from jax.experimental.pallas import tpu as pltpu
from jax.experimental.pallas import tpu_sc as plsc

VOCAB = 100000
DIM = 128
NUM_CLASSES = 2
L = 16384

_info = plsc.get_sparse_core_info()
NC, NS, LANES = _info.num_cores, _info.num_subcores, _info.num_lanes
NW = NC * NS
PER_W = L // NW
CHUNK = 64
CHUNKS = PER_W // CHUNK
VPR = DIM // LANES


RU = 4


def _gather_sum_body(xr_hbm, table_hbm, out_hbm, idx_v, rows_v, acc_v,
                     shared_v, *sems):
    sid = lax.axis_index("s")
    cid = lax.axis_index("c")
    wid = sid * NC + cid
    icp = pltpu.async_copy(xr_hbm.at[wid], idx_v, sems[CHUNKS])
    zero = jnp.zeros((LANES,), jnp.float32)

    def zbody(r, c):
        for k in range(VPR):
            rows_v[r, pl.ds(k * LANES, LANES)] = zero
        return c

    plsc.parallel_loop(0, CHUNK, carry=(jnp.int32(0),), unroll=RU)(
        lambda r, c: (zbody(r, c[0]),))
    icp.wait()

    cps = [
        pltpu.async_copy(table_hbm.at[idx_v.at[j]], rows_v, sems[j], add=True)
        for j in range(CHUNKS)
    ]
    for j in range(CHUNKS):
        cps[j].wait()

    accs = tuple(jnp.zeros((LANES,), jnp.float32) for _ in range(VPR))

    def body(r, a):
        return tuple(a[k] + rows_v[r, pl.ds(k * LANES, LANES)]
                     for k in range(VPR))

    accs = plsc.parallel_loop(0, CHUNK, carry=accs, unroll=RU)(body)

    for k in range(VPR):
        acc_v[pl.ds(k * LANES, LANES)] = accs[k]

    pltpu.sync_copy(acc_v, shared_v.at[sid])
    plsc.subcore_barrier()

    @pl.when(sid == 0)
    def _():
        pltpu.sync_copy(shared_v, rows_v.at[pl.ds(0, NS)])

        def rbody(r, a):
            return tuple(a[k] + rows_v[r, pl.ds(k * LANES, LANES)]
                         for k in range(VPR))

        accs2 = plsc.parallel_loop(
            0, NS,
            carry=tuple(jnp.zeros((LANES,), jnp.float32) for _ in range(VPR)),
            unroll=RU)(rbody)
        for k in range(VPR):
            acc_v[pl.ds(k * LANES, LANES)] = accs2[k]
        pltpu.sync_copy(acc_v, out_hbm.at[cid])


_gather_sum = functools.partial(
    pl.kernel,
    out_type=jax.ShapeDtypeStruct((NC, DIM), jnp.float32),
    mesh=plsc.VectorSubcoreMesh(core_axis_name="c", subcore_axis_name="s"),
    scratch_types=[
        pltpu.VMEM((CHUNKS, CHUNK), jnp.int32),
        pltpu.VMEM((CHUNK, DIM), jnp.float32),
        pltpu.VMEM((DIM,), jnp.float32),
        pltpu.VMEM_SHARED((NS, DIM), jnp.float32),
    ] + [pltpu.SemaphoreType.DMA] * (CHUNKS + 1),
)(_gather_sum_body)


def _tail_body(p_ref, w_ref, b_ref, o_ref):
    s = jnp.sum(p_ref[...], axis=0, keepdims=True) * (1.0 / L)
    h = jnp.maximum(s, 0.0)
    logits = lax.dot_general(h, w_ref[...], (((1,), (1,)), ((), ())))
    logits = logits + b_ref[...]
    mx = jnp.max(logits, axis=1, keepdims=True)
    lse = mx + jnp.log(jnp.sum(jnp.exp(logits - mx), axis=1, keepdims=True))
    o_ref[...] = logits - lse


_tail = pl.pallas_call(
    _tail_body,
    out_shape=jax.ShapeDtypeStruct((1, NUM_CLASSES), jnp.float32),
)


def kernel(x, emb_table, W, b):
    xr = x.reshape(NW, CHUNKS, CHUNK).astype(jnp.int32)
    partials = _gather_sum(xr, emb_table)
    return _tail(partials, W, b.reshape(1, NUM_CLASSES))


# speedup vs baseline: 1.0141x; 1.0141x over previous
import functools

import jax
import jax.numpy as jnp
from jax import lax
from jax.experimental import pallas as pl
from jax.experimental.pallas import tpu as pltpu
from jax.experimental.pallas import tpu_sc as plsc

VOCAB = 100000
DIM = 128
NUM_CLASSES = 2
L = 16384

_info = plsc.get_sparse_core_info()
NC, NS, LANES = _info.num_cores, _info.num_subcores, _info.num_lanes
NW = NC * NS
PER_W = L // NW
CHUNK = 64
CHUNKS = PER_W // CHUNK
VPR = DIM // LANES


RU = 4


def _gather_sum_body(xr_hbm, table_hbm, out_hbm, idx_v, rows_v, acc_v, *sems):
    wid = lax.axis_index("s") * NC + lax.axis_index("c")
    icp = pltpu.async_copy(xr_hbm.at[wid], idx_v, sems[CHUNKS])
    zero = jnp.zeros((LANES,), jnp.float32)

    def zbody(r, c):
        for k in range(VPR):
            rows_v[r, pl.ds(k * LANES, LANES)] = zero
        return c

    plsc.parallel_loop(0, CHUNK, carry=(jnp.int32(0),), unroll=RU)(
        lambda r, c: (zbody(r, c[0]),))
    icp.wait()

    cps = [
        pltpu.async_copy(table_hbm.at[idx_v.at[j]], rows_v, sems[j], add=True)
        for j in range(CHUNKS)
    ]
    for j in range(CHUNKS):
        cps[j].wait()

    accs = tuple(jnp.zeros((LANES,), jnp.float32) for _ in range(VPR))

    def body(r, a):
        return tuple(a[k] + rows_v[r, pl.ds(k * LANES, LANES)]
                     for k in range(VPR))

    accs = plsc.parallel_loop(0, CHUNK, carry=accs, unroll=8)(body)

    for k in range(VPR):
        acc_v[pl.ds(k * LANES, LANES)] = accs[k]

    pltpu.sync_copy(acc_v, out_hbm.at[wid])


_gather_sum = functools.partial(
    pl.kernel,
    out_type=jax.ShapeDtypeStruct((NW, DIM), jnp.float32),
    mesh=plsc.VectorSubcoreMesh(core_axis_name="c", subcore_axis_name="s"),
    scratch_types=[
        pltpu.VMEM((CHUNKS, CHUNK), jnp.int32),
        pltpu.VMEM((CHUNK, DIM), jnp.float32),
        pltpu.VMEM((DIM,), jnp.float32),
    ] + [pltpu.SemaphoreType.DMA] * (CHUNKS + 1),
)(_gather_sum_body)


def _tail_body(p_ref, w_ref, b_ref, o_ref):
    s = jnp.sum(p_ref[...], axis=0, keepdims=True) * (1.0 / L)
    h = jnp.maximum(s, 0.0)
    logits = lax.dot_general(h, w_ref[...], (((1,), (1,)), ((), ())))
    logits = logits + b_ref[...]
    mx = jnp.max(logits, axis=1, keepdims=True)
    lse = mx + jnp.log(jnp.sum(jnp.exp(logits - mx), axis=1, keepdims=True))
    o_ref[...] = logits - lse


_tail = pl.pallas_call(
    _tail_body,
    out_shape=jax.ShapeDtypeStruct((1, NUM_CLASSES), jnp.float32),
)


def kernel(x, emb_table, W, b):
    xr = x.reshape(NW, CHUNKS, CHUNK).astype(jnp.int32)
    partials = _gather_sum(xr, emb_table)
    return _tail(partials, W, b.reshape(1, NUM_CLASSES))
